# baseline (speedup 1.0000x reference)
import functools

import jax
import jax.numpy as jnp
from jax import lax
from jax.experimental import pallas as pl
from jax.experimental.pallas import tpu as pltpu
from jax.experimental.pallas import tpu_sc as plsc

HIDDEN = 1024
NC = 2
NS = 16
NW = NC * NS


def _make_gather(ntok: int):
    assert ntok % NW == 0
    bpw = ntok // NW

    mesh = plsc.VectorSubcoreMesh(core_axis_name="c", subcore_axis_name="s")

    @functools.partial(
        pl.kernel,
        mesh=mesh,
        out_type=jax.ShapeDtypeStruct((ntok, HIDDEN), jnp.float32),
        scratch_types=[
            pltpu.VMEM((bpw,), jnp.int32),
            pltpu.SemaphoreType.DMA,
            pltpu.SemaphoreType.DMA,
        ],
    )
    def gather_kernel(ids_hbm, table_hbm, out_hbm, idx_v, isem, dsem):
        wid = lax.axis_index("s") * NC + lax.axis_index("c")
        base = wid * bpw
        pltpu.sync_copy(ids_hbm.at[pl.ds(base, bpw)], idx_v)

        @pl.loop(0, bpw // 16)
        def _(j):
            vec = idx_v[pl.ds(j * 16, 16)]
            for lane in range(16):
                row = vec[lane]
                pltpu.async_copy(
                    table_hbm.at[pl.ds(row, 1)],
                    out_hbm.at[pl.ds(base + j * 16 + lane, 1)],
                    dsem,
                )

        pltpu.make_async_copy(
            table_hbm.at[pl.ds(0, bpw)], out_hbm.at[pl.ds(base, bpw)], dsem
        ).wait()

    return gather_kernel


def kernel(input_ids, position_ids, embed_tokens):
    batch, seq = input_ids.shape
    ids_flat = input_ids.reshape(-1)
    rows = _make_gather(batch * seq)(ids_flat, embed_tokens)
    hidden_states = rows.reshape(batch, seq, HIDDEN)
    return hidden_states, position_ids


# speedup vs baseline: 35.8641x; 35.8641x over previous
import functools

import jax
import jax.numpy as jnp
from jax import lax
from jax.experimental import pallas as pl
from jax.experimental.pallas import tpu as pltpu
from jax.experimental.pallas import tpu_sc as plsc

HIDDEN = 1024
NC = 2
NS = 16
NW = NC * NS
CHUNK = 32


def _make_gather(ntok: int):
    assert ntok % NW == 0
    bpw = ntok // NW
    assert bpw % CHUNK == 0
    nch = bpw // CHUNK
    nbuf = 2
    assert nch % nbuf == 0

    mesh = plsc.VectorSubcoreMesh(core_axis_name="c", subcore_axis_name="s")

    @functools.partial(
        pl.kernel,
        mesh=mesh,
        out_type=jax.ShapeDtypeStruct((ntok, HIDDEN), jnp.float32),
        scratch_types=[
            pltpu.VMEM((bpw,), jnp.int32),
            [pltpu.VMEM((CHUNK, HIDDEN), jnp.float32) for _ in range(nbuf)],
            [pltpu.SemaphoreType.DMA for _ in range(nbuf)],
            [pltpu.SemaphoreType.DMA for _ in range(nbuf)],
        ],
    )
    def gather_kernel(ids_hbm, table_hbm, out_hbm, idx_v, bufs, gsems, ssems):
        wid = lax.axis_index("s") * NC + lax.axis_index("c")
        base = wid * bpw
        pltpu.sync_copy(ids_hbm.at[pl.ds(base, bpw)], idx_v)

        def start_gather(ch, b):
            pltpu.async_copy(
                table_hbm.at[idx_v.at[pl.ds(ch * CHUNK, CHUNK)]], bufs[b], gsems[b]
            )

        for b in range(nbuf):
            start_gather(b, b)

        @pl.loop(0, nch, step=nbuf)
        def _(i):
            for b in range(nbuf):
                ch = i + b
                pltpu.make_async_copy(
                    table_hbm.at[idx_v.at[pl.ds(0, CHUNK)]], bufs[b], gsems[b]
                ).wait()
                st = pltpu.async_copy(
                    bufs[b], out_hbm.at[pl.ds(base + ch * CHUNK, CHUNK)], ssems[b]
                )
                st.wait()

                @pl.when(ch + nbuf < nch)
                def _():
                    start_gather(ch + nbuf, b)

    return gather_kernel


def kernel(input_ids, position_ids, embed_tokens):
    batch, seq = input_ids.shape
    ids_flat = input_ids.reshape(-1)
    rows = _make_gather(batch * seq)(ids_flat, embed_tokens)
    hidden_states = rows.reshape(batch, seq, HIDDEN)
    return hidden_states, position_ids
